# baseline (speedup 1.0000x reference)
import jax
import jax.numpy as jnp
from jax.experimental import pallas as pl
from jax.experimental.pallas import tpu as pltpu
from jax.experimental.pallas import tpu_sc as plsc

BATCH = 4096
N_FIELDS = 26
EMBED_DIM = 64
NUM_INDICES = BATCH * N_FIELDS
WINDOW = 832


def kernel(indices, table):
    idx_flat = indices.reshape(1, NUM_INDICES).astype(jnp.int32)

    mesh = plsc.VectorSubcoreMesh(core_axis_name="core", subcore_axis_name="subcore")

    @pl.kernel(
        out_type=jax.ShapeDtypeStruct((NUM_INDICES, EMBED_DIM), table.dtype),
        mesh=mesh,
        compiler_params=pltpu.CompilerParams(use_tc_tiling_on_sc=False),
    )
    def gather_kernel(table_hbm, idx_hbm, out_hbm):
        def body(idx_vmem, out_vmem):
            pltpu.sync_copy(table_hbm.at[idx_vmem.at[0]], out_vmem)

        pltpu.emit_pipeline(
            body,
            grid=(NUM_INDICES // WINDOW,),
            in_specs=[pl.BlockSpec((1, WINDOW), index_map=lambda i: (0, i))],
            out_specs=[
                pl.BlockSpec((WINDOW, EMBED_DIM), index_map=lambda i: (i, 0))
            ],
            core_axis_name=("core", "subcore"),
            dimension_semantics=(pltpu.PARALLEL,),
        )(idx_hbm, out_hbm)

    out = gather_kernel(table, idx_flat)
    return out.reshape(BATCH, N_FIELDS, EMBED_DIM)


# speedup vs baseline: 1.0034x; 1.0034x over previous
import functools

import jax
import jax.numpy as jnp
from jax import lax
from jax.experimental import pallas as pl
from jax.experimental.pallas import tpu as pltpu
from jax.experimental.pallas import tpu_sc as plsc

BATCH = 4096
N_FIELDS = 26
EMBED_DIM = 64
B = BATCH * N_FIELDS
NW = 32
BPW = B // NW
CH = 128
NCH = BPW // CH
NBUF = 8


def kernel(indices, table):
    idx_flat = indices.reshape(NW, NCH, CH).astype(jnp.int32)

    mesh = plsc.VectorSubcoreMesh(core_axis_name="c", subcore_axis_name="s")

    @functools.partial(
        pl.kernel,
        out_type=jax.ShapeDtypeStruct((B, EMBED_DIM), jnp.float32),
        mesh=mesh,
        scratch_types=[
            pltpu.VMEM((NCH, CH), jnp.int32),
            pltpu.VMEM((NBUF, CH, EMBED_DIM), jnp.float32),
            pltpu.SemaphoreType.DMA,
            pltpu.SemaphoreType.DMA((NBUF,)),
            pltpu.SemaphoreType.DMA((NBUF,)),
        ],
        compiler_params=pltpu.CompilerParams(use_tc_tiling_on_sc=False),
    )
    def gather_kernel(table_hbm, idx_hbm, out_hbm, idx_v, rows_v, isem, gsem, wsem):
        wid = lax.axis_index("s") * 2 + lax.axis_index("c")
        base = wid * BPW
        cp = pltpu.make_async_copy(idx_hbm.at[wid], idx_v, isem)
        cp.start()
        cp.wait()

        def gather_cp(c, b):
            return pltpu.make_async_copy(
                table_hbm.at[idx_v.at[c]], rows_v.at[b], gsem.at[b]
            )

        def write_cp(c, b):
            return pltpu.make_async_copy(
                rows_v.at[b], out_hbm.at[pl.ds(base + c * CH, CH)], wsem.at[b]
            )

        for b in range(NBUF):
            gather_cp(b, b).start()

        for c in range(NCH):
            b = c % NBUF
            gather_cp(c, b).wait()
            write_cp(c, b).start()
            n = c + NBUF
            if n < NCH:
                write_cp(c, b).wait()
                gather_cp(n, b).start()

        for c in range(NCH - NBUF, NCH):
            b = c % NBUF
            write_cp(c, b).wait()

    out = gather_kernel(table, idx_flat)
    return out.reshape(BATCH, N_FIELDS, EMBED_DIM)
